# baseline (speedup 1.0000x reference)
import functools

import jax
import jax.numpy as jnp
from jax import lax
from jax.experimental import pallas as pl
from jax.experimental.pallas import tpu as pltpu
from jax.experimental.pallas import tpu_sc as plsc

_NC = 2
_NS = 16
_NW = _NC * _NS
_LANES = 16
_CH = 16
_UNROLL = 8


def kernel(x, pos_embedding):
    b, s, d = x.shape
    rows_per_w = s // _NW
    n_chunks = rows_per_w // _CH
    n_steps = n_chunks * b
    steps_per_iter = 2 * b
    n_iters = n_steps // steps_per_iter
    groups = d // _LANES
    mesh = plsc.VectorSubcoreMesh(core_axis_name="c", subcore_axis_name="s")

    @functools.partial(
        pl.kernel,
        mesh=mesh,
        out_type=jax.ShapeDtypeStruct((b, s, d), jnp.float32),
        scratch_types=[
            pltpu.VMEM((2, _CH, d), jnp.float32),
            pltpu.VMEM((4, _CH, d), jnp.float32),
            pltpu.SemaphoreType.DMA((4,)),
            pltpu.SemaphoreType.DMA((4,)),
            pltpu.SemaphoreType.DMA((2,)),
        ],
    )
    def k(x_hbm, pos_hbm, out_hbm, pos_v, x_v, sem_in, sem_out, sem_pos):
        wid = lax.axis_index("s") * _NC + lax.axis_index("c")
        row_base = wid * rows_per_w

        def coords(cp, t):
            return t % b, row_base + (cp * 2 + t // b) * _CH

        pltpu.async_copy(pos_hbm.at[pl.ds(row_base, _CH)], pos_v.at[0],
                         sem_pos.at[0])
        pltpu.async_copy(x_hbm.at[0, pl.ds(row_base, _CH)], x_v.at[0],
                         sem_in.at[0])
        pltpu.async_copy(x_hbm.at[1, pl.ds(row_base, _CH)], x_v.at[1],
                         sem_in.at[1])

        def iter_body(cp, _):
            r0 = row_base + cp * 2 * _CH

            for u in range(steps_per_iter):
                k_, bi = u // b, u % b
                xb = u % 4
                pb = k_
                rows = r0 + k_ * _CH

                if u == 0 or u == b:
                    pltpu.async_copy(
                        pos_hbm.at[pl.ds(rows + _CH, _CH)],
                        pos_v.at[(pb + 1) % 2], sem_pos.at[(pb + 1) % 2])

                nb = (u + 2) % 4
                bp, rp = coords(cp, u - 2)
                bn, rn = coords(cp, u + 2)
                if u < 2:
                    @pl.when(cp > 0)
                    def _():
                        pltpu.make_async_copy(
                            x_v.at[nb], out_hbm.at[bp, pl.ds(rp, _CH)],
                            sem_out.at[nb]).wait()
                else:
                    pltpu.make_async_copy(
                        x_v.at[nb], out_hbm.at[bp, pl.ds(rp, _CH)],
                        sem_out.at[nb]).wait()
                if u < steps_per_iter - 2:
                    pltpu.async_copy(
                        x_hbm.at[bn, pl.ds(rn, _CH)], x_v.at[nb],
                        sem_in.at[nb])
                else:
                    @pl.when(cp + 1 < n_iters)
                    def _():
                        pltpu.async_copy(
                            x_hbm.at[bn, pl.ds(rn, _CH)], x_v.at[nb],
                            sem_in.at[nb])

                pltpu.make_async_copy(
                    x_hbm.at[bi, pl.ds(rows, _CH)], x_v.at[xb],
                    sem_in.at[xb]).wait()
                if bi == 0:
                    pltpu.make_async_copy(
                        pos_hbm.at[pl.ds(rows, _CH)], pos_v.at[pb],
                        sem_pos.at[pb]).wait()

                def row_body(r, _):
                    def col_body(j, _):
                        for v in range(_UNROLL):
                            o = (j * _UNROLL + v) * _LANES
                            x_v[xb, r, pl.ds(o, _LANES)] = (
                                x_v[xb, r, pl.ds(o, _LANES)]
                                + pos_v[pb, r, pl.ds(o, _LANES)])
                        return 0

                    lax.fori_loop(0, groups // _UNROLL, col_body, 0,
                                  unroll=True)
                    return 0

                q_rows = _CH // 4

                def quarter_body(q, _):
                    lax.fori_loop(q * q_rows, (q + 1) * q_rows, row_body, 0)
                    pltpu.async_copy(
                        x_v.at[xb, pl.ds(q * q_rows, q_rows)],
                        out_hbm.at[bi, pl.ds(rows + q * q_rows, q_rows)],
                        sem_out.at[xb])
                    return 0

                lax.fori_loop(0, 4, quarter_body, 0)
            return 0

        lax.fori_loop(0, n_iters, iter_body, 0)

        for t in (n_steps - 2, n_steps - 1):
            bi, rows = t % b, row_base + (t // b) * _CH
            pltpu.make_async_copy(
                x_v.at[t % 4], out_hbm.at[bi, pl.ds(rows, _CH)],
                sem_out.at[t % 4]).wait()
        pltpu.make_async_copy(
            pos_hbm.at[pl.ds(row_base, _CH)], pos_v.at[0],
            sem_pos.at[0]).wait()

    return k(x, pos_embedding)


# speedup vs baseline: 1.7020x; 1.7020x over previous
import functools

import jax
import jax.numpy as jnp
from jax import lax
from jax.experimental import pallas as pl
from jax.experimental.pallas import tpu as pltpu
from jax.experimental.pallas import tpu_sc as plsc

_NC = 2
_NS = 16
_NW = _NC * _NS
_LANES = 16
_CH = 16
_UNROLL = 8


def kernel(x, pos_embedding):
    b, s, d = x.shape
    rows_per_w = s // _NW
    n_chunks = rows_per_w // _CH
    n_steps = n_chunks * b
    steps_per_iter = 2 * b
    n_iters = n_steps // steps_per_iter
    groups = d // _LANES
    mesh = plsc.VectorSubcoreMesh(core_axis_name="c", subcore_axis_name="s")

    @functools.partial(
        pl.kernel,
        mesh=mesh,
        out_type=jax.ShapeDtypeStruct((b, s, d), jnp.float32),
        scratch_types=[
            pltpu.VMEM((2, _CH, d), jnp.float32),
            pltpu.VMEM((4, _CH, d), jnp.float32),
            pltpu.SemaphoreType.DMA((4,)),
            pltpu.SemaphoreType.DMA((4,)),
            pltpu.SemaphoreType.DMA((2,)),
        ],
    )
    def k(x_hbm, pos_hbm, out_hbm, pos_v, x_v, sem_in, sem_out, sem_pos):
        wid = lax.axis_index("s") * _NC + lax.axis_index("c")
        row_base = wid * rows_per_w

        def coords(cp, t):
            return t % b, row_base + (cp * 2 + t // b) * _CH

        pltpu.async_copy(pos_hbm.at[pl.ds(row_base, _CH)], pos_v.at[0],
                         sem_pos.at[0])
        pltpu.async_copy(x_hbm.at[0, pl.ds(row_base, _CH)], x_v.at[0],
                         sem_in.at[0])
        pltpu.async_copy(x_hbm.at[1, pl.ds(row_base, _CH)], x_v.at[1],
                         sem_in.at[1])

        def iter_body(cp, _):
            r0 = row_base + cp * 2 * _CH

            for u in range(steps_per_iter):
                k_, bi = u // b, u % b
                xb = u % 4
                pb = k_
                rows = r0 + k_ * _CH

                if u == 0 or u == b:
                    pltpu.async_copy(
                        pos_hbm.at[pl.ds(rows + _CH, _CH)],
                        pos_v.at[(pb + 1) % 2], sem_pos.at[(pb + 1) % 2])

                nb = (u + 2) % 4
                bp, rp = coords(cp, u - 2)
                bn, rn = coords(cp, u + 2)
                if u < 2:
                    @pl.when(cp > 0)
                    def _():
                        pltpu.make_async_copy(
                            x_v.at[nb], out_hbm.at[bp, pl.ds(rp, _CH)],
                            sem_out.at[nb]).wait()
                else:
                    pltpu.make_async_copy(
                        x_v.at[nb], out_hbm.at[bp, pl.ds(rp, _CH)],
                        sem_out.at[nb]).wait()
                if u < steps_per_iter - 2:
                    pltpu.async_copy(
                        x_hbm.at[bn, pl.ds(rn, _CH)], x_v.at[nb],
                        sem_in.at[nb])
                else:
                    @pl.when(cp + 1 < n_iters)
                    def _():
                        pltpu.async_copy(
                            x_hbm.at[bn, pl.ds(rn, _CH)], x_v.at[nb],
                            sem_in.at[nb])

                pltpu.make_async_copy(
                    x_hbm.at[bi, pl.ds(rows, _CH)], x_v.at[xb],
                    sem_in.at[xb]).wait()
                if bi == 0:
                    pltpu.make_async_copy(
                        pos_hbm.at[pl.ds(rows, _CH)], pos_v.at[pb],
                        sem_pos.at[pb]).wait()

                def row_body(r, _):
                    def col_body(j, _):
                        for v in range(_UNROLL):
                            o = (j * _UNROLL + v) * _LANES
                            x_v[xb, r, pl.ds(o, _LANES)] = (
                                x_v[xb, r, pl.ds(o, _LANES)]
                                + pos_v[pb, r, pl.ds(o, _LANES)])
                        return 0

                    lax.fori_loop(0, groups // _UNROLL, col_body, 0,
                                  unroll=True)
                    return 0

                h_rows = _CH // 2
                for h in range(2):
                    lax.fori_loop(h * h_rows, (h + 1) * h_rows, row_body, 0)
                    pltpu.async_copy(
                        x_v.at[xb, pl.ds(h * h_rows, h_rows)],
                        out_hbm.at[bi, pl.ds(rows + h * h_rows, h_rows)],
                        sem_out.at[xb])
            return 0

        lax.fori_loop(0, n_iters, iter_body, 0)

        for t in (n_steps - 2, n_steps - 1):
            bi, rows = t % b, row_base + (t // b) * _CH
            pltpu.make_async_copy(
                x_v.at[t % 4], out_hbm.at[bi, pl.ds(rows, _CH)],
                sem_out.at[t % 4]).wait()
        pltpu.make_async_copy(
            pos_hbm.at[pl.ds(row_base, _CH)], pos_v.at[0],
            sem_pos.at[0]).wait()

    return k(x, pos_embedding)


# speedup vs baseline: 2.6225x; 1.5409x over previous
import functools

import jax
import jax.numpy as jnp
from jax import lax
from jax.experimental import pallas as pl
from jax.experimental.pallas import tpu as pltpu
from jax.experimental.pallas import tpu_sc as plsc

_NC = 2
_NS = 16
_NW = _NC * _NS
_LANES = 16
_CH = 8
_HALF = 32


def kernel(x, pos_embedding):
    b, s, d = x.shape
    rows_per_w = s // _NW
    n_chunks = rows_per_w // _CH
    n_iters = n_chunks // 2
    groups = d // _LANES
    mesh = plsc.VectorSubcoreMesh(core_axis_name="c", subcore_axis_name="s")

    @functools.partial(
        pl.kernel,
        mesh=mesh,
        out_type=jax.ShapeDtypeStruct((b, s, d), jnp.float32),
        scratch_types=[
            pltpu.VMEM((2, _CH, d), jnp.float32),
            pltpu.VMEM((2, b, _CH, d), jnp.float32),
            pltpu.SemaphoreType.DMA((2,)),
            pltpu.SemaphoreType.DMA((2,)),
            pltpu.SemaphoreType.DMA((2,)),
        ],
    )
    def k(x_hbm, pos_hbm, out_hbm, pos_v, x_v, sem_in, sem_out, sem_pos):
        wid = lax.axis_index("s") * _NC + lax.axis_index("c")
        row_base = wid * rows_per_w

        pltpu.async_copy(pos_hbm.at[pl.ds(row_base, _CH)], pos_v.at[0],
                         sem_pos.at[0])
        pltpu.async_copy(x_hbm.at[:, pl.ds(row_base, _CH)], x_v.at[0],
                         sem_in.at[0])

        def iter_body(cp, _):
            for k_ in range(2):
                xb = k_
                ob = (k_ + 1) % 2
                rows = row_base + (cp * 2 + k_) * _CH
                nxt = rows + _CH

                pltpu.async_copy(pos_hbm.at[pl.ds(nxt, _CH)],
                                 pos_v.at[ob], sem_pos.at[ob])

                if k_ == 0:
                    @pl.when(cp > 0)
                    def _():
                        pltpu.make_async_copy(
                            x_v.at[ob],
                            out_hbm.at[:, pl.ds(rows - _CH, _CH)],
                            sem_out.at[ob]).wait()
                    pltpu.async_copy(x_hbm.at[:, pl.ds(nxt, _CH)],
                                     x_v.at[ob], sem_in.at[ob])
                else:
                    pltpu.make_async_copy(
                        x_v.at[ob], out_hbm.at[:, pl.ds(rows - _CH, _CH)],
                        sem_out.at[ob]).wait()

                    @pl.when(cp + 1 < n_iters)
                    def _():
                        pltpu.async_copy(x_hbm.at[:, pl.ds(nxt, _CH)],
                                         x_v.at[ob], sem_in.at[ob])

                pltpu.make_async_copy(
                    x_hbm.at[:, pl.ds(rows, _CH)], x_v.at[xb],
                    sem_in.at[xb]).wait()
                pltpu.make_async_copy(
                    pos_hbm.at[pl.ds(rows, _CH)], pos_v.at[xb],
                    sem_pos.at[xb]).wait()

                def row_body(r, _):
                    for h in range(groups // _HALF):
                        base = h * _HALF * _LANES
                        pos_regs = [
                            pos_v[xb, r, pl.ds(base + g * _LANES, _LANES)]
                            for g in range(_HALF)
                        ]
                        for bi in range(b):
                            for g in range(_HALF):
                                o = base + g * _LANES
                                x_v[xb, bi, r, pl.ds(o, _LANES)] = (
                                    x_v[xb, bi, r, pl.ds(o, _LANES)]
                                    + pos_regs[g])
                    return 0

                hc = _CH // 2
                for hh in range(2):
                    lax.fori_loop(hh * hc, (hh + 1) * hc, row_body, 0)
                    pltpu.async_copy(
                        x_v.at[xb, :, pl.ds(hh * hc, hc)],
                        out_hbm.at[:, pl.ds(rows + hh * hc, hc)],
                        sem_out.at[xb])
            return 0

        lax.fori_loop(0, n_iters, iter_body, 0)

        last = row_base + (n_chunks - 1) * _CH
        pltpu.make_async_copy(
            x_v.at[1], out_hbm.at[:, pl.ds(last, _CH)], sem_out.at[1]).wait()
        pltpu.make_async_copy(
            pos_hbm.at[pl.ds(row_base, _CH)], pos_v.at[0],
            sem_pos.at[0]).wait()

    return k(x, pos_embedding)
